# Initial kernel scaffold; baseline (speedup 1.0000x reference)
import jax
import jax.numpy as jnp
from jax.experimental import pallas as pl


def kernel(features, sparse_adj, W_gcn, b_gcn, Fmat, embeddings):
    raise NotImplementedError("write your pallas kernel here")


# speedup vs baseline: 23.4074x; 23.4074x over previous
import functools

import jax
import jax.numpy as jnp
from jax import lax
from jax.experimental import pallas as pl
from jax.experimental.pallas import tpu as pltpu
from jax.experimental.pallas import tpu_sc as plsc

_N = 10000
_D = 128
_E = 320000
_NP = 10240
_CH = 80
_ROWS = _E // _CH
_NSUB = 16
_NSTG = 5
_SROWS = _ROWS // _NSUB // _NSTG
_SLICE = _NP // _NSUB
_KAPPA = 0.95

_mesh = plsc.VectorSubcoreMesh(core_axis_name="c", subcore_axis_name="s")


def _deg_body(dst4d, zrow, deg_out, dstb, ones_v, deg_sh, sem):
    c = lax.axis_index("c")
    s = lax.axis_index("s")
    for k in range(_CH // 16):
        ones_v[pl.ds(16 * k, 16)] = jnp.full((16,), 1.0, jnp.float32)
    pltpu.sync_copy(zrow, deg_sh.at[pl.ds(s * _SLICE, _SLICE)])
    for st in range(_NSTG):
        pltpu.sync_copy(dst4d.at[s, st], dstb.at[st])
    plsc.subcore_barrier()
    half = _SROWS // 2

    def grp(g, carry):
        st = g // (half // 5)
        gg = g % (half // 5)
        descs = [
            pltpu.async_copy(
                ones_v, deg_sh.at[dstb.at[st, c * half + gg * 5 + b]], sem,
                add=True,
            )
            for b in range(5)
        ]
        for d in descs:
            d.wait()
        return carry

    lax.fori_loop(0, _NSTG * (half // 5), grp, 0)
    plsc.subcore_barrier()
    pltpu.sync_copy(deg_sh.at[pl.ds(s * _SLICE, _SLICE)],
                    deg_out.at[c, pl.ds(s * _SLICE, _SLICE)])


_deg_call = functools.partial(
    pl.kernel,
    out_type=jax.ShapeDtypeStruct((2, _NP), jnp.float32),
    mesh=_mesh,
    scratch_types=[
        pltpu.VMEM((_NSTG, _SROWS, _CH), jnp.int32),
        pltpu.VMEM((_CH,), jnp.float32),
        pltpu.VMEM_SHARED((_NP,), jnp.float32),
        pltpu.SemaphoreType.DMA,
    ],
)(_deg_body)


_NB = 2


def _seg_body(h2_t, e2_t, src4d, dst4d, s1_out, s2_out,
              srcb, dstb, rows, acc_sh, g0, g1, s0, s1sem):
    c = lax.axis_index("c")
    s = lax.axis_index("s")
    gsems = [g0, g1]
    ssems = [s0, s1sem]

    def zr(r, carry):
        for k in range(_D // 16):
            rows[0, r, pl.ds(16 * k, 16)] = jnp.zeros((16,), jnp.float32)
        return carry

    lax.fori_loop(0, _CH, zr, 0)
    for k in range(_SLICE // _CH):
        pltpu.sync_copy(rows.at[0],
                        acc_sh.at[pl.ds(s * _SLICE + k * _CH, _CH)])
    plsc.subcore_barrier()

    def make_stage(table):
        def stage(st, carry):
            pltpu.sync_copy(src4d.at[s, st], srcb)
            pltpu.sync_copy(dst4d.at[s, st], dstb)

            def grp(g, carry2):
                gd = []
                for b in range(_NB):
                    @pl.when(g > 0)
                    def _(b=b):
                        pltpu.make_async_copy(
                            table.at[pl.ds(0, _CH)], rows.at[b], ssems[b]
                        ).wait()
                    gd.append(
                        pltpu.async_copy(
                            table.at[srcb.at[g * _NB + b]], rows.at[b], gsems[b]
                        )
                    )
                for b in range(_NB):
                    gd[b].wait()
                    pltpu.async_copy(
                        rows.at[b], acc_sh.at[dstb.at[g * _NB + b]], ssems[b],
                        add=True,
                    )
                return carry2

            lax.fori_loop(0, _SROWS // _NB, grp, 0)
            for b in range(_NB):
                pltpu.make_async_copy(
                    table.at[pl.ds(0, _CH)], rows.at[b], ssems[b]
                ).wait()
            return carry

        return stage

    @pl.when(c == 0)
    def _():
        lax.fori_loop(0, _NSTG, make_stage(h2_t), 0)

    @pl.when(c == 1)
    def _():
        lax.fori_loop(0, _NSTG, make_stage(e2_t), 0)

    plsc.subcore_barrier()

    @pl.when(c == 0)
    def _():
        pltpu.sync_copy(acc_sh.at[pl.ds(s * _SLICE, _SLICE)],
                        s1_out.at[pl.ds(s * _SLICE, _SLICE)])

    @pl.when(c == 1)
    def _():
        pltpu.sync_copy(acc_sh.at[pl.ds(s * _SLICE, _SLICE)],
                        s2_out.at[pl.ds(s * _SLICE, _SLICE)])


_seg_call = functools.partial(
    pl.kernel,
    out_type=(
        jax.ShapeDtypeStruct((_NP, _D), jnp.float32),
        jax.ShapeDtypeStruct((_NP, _D), jnp.float32),
    ),
    mesh=_mesh,
    scratch_types=[
        pltpu.VMEM((_SROWS, _CH), jnp.int32),
        pltpu.VMEM((_SROWS, _CH), jnp.int32),
        pltpu.VMEM((_NB, _CH, _D), jnp.float32),
        pltpu.VMEM_SHARED((_NP, _D), jnp.float32),
        pltpu.SemaphoreType.DMA,
        pltpu.SemaphoreType.DMA,
        pltpu.SemaphoreType.DMA,
        pltpu.SemaphoreType.DMA,
    ],
)(_seg_body)


_BR = 1000


def _dense_body(feat, emb, wg, fm, d0, d1, h2, e2):
    di = lax.rsqrt(d0[...] + d1[...] + 1.0)
    h = jnp.dot(feat[...], wg[...], preferred_element_type=jnp.float32)
    h2[...] = h * di
    m = lax.dot_general(fm[...], fm[...], (((0,), (0,)), ((), ())),
                        preferred_element_type=jnp.float32)
    nrm = jnp.sqrt(jnp.sum(m * m)) + 1e-5
    e2[...] = jnp.dot(emb[...], m, preferred_element_type=jnp.float32) * (_KAPPA / nrm)


def _dense_call(features, embeddings, W_gcn, Fmat, d0, d1):
    return pl.pallas_call(
        _dense_body,
        grid=(_N // _BR,),
        in_specs=[
            pl.BlockSpec((_BR, _D), lambda i: (i, 0)),
            pl.BlockSpec((_BR, _D), lambda i: (i, 0)),
            pl.BlockSpec((_D, _D), lambda i: (0, 0)),
            pl.BlockSpec((_D, _D), lambda i: (0, 0)),
            pl.BlockSpec((_BR, 1), lambda i: (i, 0)),
            pl.BlockSpec((_BR, 1), lambda i: (i, 0)),
        ],
        out_specs=[
            pl.BlockSpec((_BR, _D), lambda i: (i, 0)),
            pl.BlockSpec((_BR, _D), lambda i: (i, 0)),
        ],
        out_shape=[
            jax.ShapeDtypeStruct((_N, _D), jnp.float32),
            jax.ShapeDtypeStruct((_N, _D), jnp.float32),
        ],
    )(features, embeddings, W_gcn, Fmat, d0, d1)


def _comb_body(feat, wg, d0, d1, s1, s2, b, out):
    di = lax.rsqrt(d0[...] + d1[...] + 1.0)
    h = jnp.dot(feat[...], wg[...], preferred_element_type=jnp.float32)
    out[...] = di * s1[...] + s2[...] + (di * di) * h + b[...]


def _comb_call(features, W_gcn, d0, d1, s1, s2, b2d):
    return pl.pallas_call(
        _comb_body,
        grid=(_N // _BR,),
        in_specs=[
            pl.BlockSpec((_BR, _D), lambda i: (i, 0)),
            pl.BlockSpec((_D, _D), lambda i: (0, 0)),
            pl.BlockSpec((_BR, 1), lambda i: (i, 0)),
            pl.BlockSpec((_BR, 1), lambda i: (i, 0)),
            pl.BlockSpec((_BR, _D), lambda i: (i, 0)),
            pl.BlockSpec((_BR, _D), lambda i: (i, 0)),
            pl.BlockSpec((1, _D), lambda i: (0, 0)),
        ],
        out_specs=pl.BlockSpec((_BR, _D), lambda i: (i, 0)),
        out_shape=jax.ShapeDtypeStruct((_N, _D), jnp.float32),
    )(features, W_gcn, d0, d1, s1, s2, b2d)


def kernel(features, sparse_adj, W_gcn, b_gcn, Fmat, embeddings):
    src4d = sparse_adj[0].astype(jnp.int32).reshape(_NSUB, _NSTG, _SROWS, _CH)
    dst4d = sparse_adj[1].astype(jnp.int32).reshape(_NSUB, _NSTG, _SROWS, _CH)
    zrow = jnp.zeros((_SLICE,), jnp.float32)

    degp = _deg_call(dst4d, zrow)
    d0 = degp[0].reshape(_NP, 1)
    d1 = degp[1].reshape(_NP, 1)
    h2, e2 = _dense_call(features, embeddings, W_gcn, Fmat, d0, d1)
    s1, s2 = _seg_call(h2, e2, src4d, dst4d)
    out = _comb_call(features, W_gcn, d0, d1, s1, s2, b_gcn.reshape(1, _D))
    return out


# speedup vs baseline: 27.5305x; 1.1761x over previous
import functools

import jax
import jax.numpy as jnp
from jax import lax
from jax.experimental import pallas as pl
from jax.experimental.pallas import tpu as pltpu
from jax.experimental.pallas import tpu_sc as plsc

_N = 10000
_D = 128
_E = 320000
_NP = 10240
_CH = 40
_ROWS = _E // _CH
_NSUB = 16
_NSTG = 10
_SROWS = _ROWS // _NSUB // _NSTG
_SLICE = _NP // _NSUB
_KAPPA = 0.95

_mesh = plsc.VectorSubcoreMesh(core_axis_name="c", subcore_axis_name="s")


def _deg_body(dst4d, zrow, deg_out, dstb, ones_v, deg_sh, sem):
    c = lax.axis_index("c")
    s = lax.axis_index("s")
    for k in range((_CH + 15) // 16):
        ones_v[pl.ds(min(16 * k, _CH - 16), 16)] = jnp.full((16,), 1.0,
                                                           jnp.float32)
    pltpu.sync_copy(zrow, deg_sh.at[pl.ds(s * _SLICE, _SLICE)])
    for st in range(_NSTG):
        pltpu.sync_copy(dst4d.at[s, st], dstb.at[st])
    plsc.subcore_barrier()
    half = _SROWS // 2

    def grp(g, carry):
        st = g // (half // 5)
        gg = g % (half // 5)
        descs = [
            pltpu.async_copy(
                ones_v, deg_sh.at[dstb.at[st, c * half + gg * 5 + b]], sem,
                add=True,
            )
            for b in range(5)
        ]
        for d in descs:
            d.wait()
        return carry

    lax.fori_loop(0, _NSTG * (half // 5), grp, 0)
    plsc.subcore_barrier()
    pltpu.sync_copy(deg_sh.at[pl.ds(s * _SLICE, _SLICE)],
                    deg_out.at[c, pl.ds(s * _SLICE, _SLICE)])


_deg_call = functools.partial(
    pl.kernel,
    out_type=jax.ShapeDtypeStruct((2, _NP), jnp.float32),
    mesh=_mesh,
    scratch_types=[
        pltpu.VMEM((_NSTG, _SROWS, _CH), jnp.int32),
        pltpu.VMEM((_CH,), jnp.float32),
        pltpu.VMEM_SHARED((_NP,), jnp.float32),
        pltpu.SemaphoreType.DMA,
    ],
)(_deg_body)


_NB = 4


def _seg_body(h2_t, e2_t, src4d, dst4d, s1_out, s2_out,
              srcb, dstb, rows, acc_sh, g0, g1, g2, g3, s0, s1sem, s2sem, s3sem):
    c = lax.axis_index("c")
    s = lax.axis_index("s")
    gsems = [g0, g1, g2, g3]
    ssems = [s0, s1sem, s2sem, s3sem]

    def zr(r, carry):
        for k in range(_D // 16):
            rows[0, r, pl.ds(16 * k, 16)] = jnp.zeros((16,), jnp.float32)
        return carry

    lax.fori_loop(0, _CH, zr, 0)
    for k in range(_SLICE // _CH):
        pltpu.sync_copy(rows.at[0],
                        acc_sh.at[pl.ds(s * _SLICE + k * _CH, _CH)])
    plsc.subcore_barrier()

    def make_stage(table):
        def stage(st, carry):
            pltpu.sync_copy(src4d.at[s, st], srcb)
            pltpu.sync_copy(dst4d.at[s, st], dstb)

            def grp(g, carry2):
                gd = []
                for b in range(_NB):
                    @pl.when(g > 0)
                    def _(b=b):
                        pltpu.make_async_copy(
                            table.at[pl.ds(0, _CH)], rows.at[b], ssems[b]
                        ).wait()
                    gd.append(
                        pltpu.async_copy(
                            table.at[srcb.at[g * _NB + b]], rows.at[b], gsems[b]
                        )
                    )
                for b in range(_NB):
                    gd[b].wait()
                    pltpu.async_copy(
                        rows.at[b], acc_sh.at[dstb.at[g * _NB + b]], ssems[b],
                        add=True,
                    )
                return carry2

            lax.fori_loop(0, _SROWS // _NB, grp, 0)
            for b in range(_NB):
                pltpu.make_async_copy(
                    table.at[pl.ds(0, _CH)], rows.at[b], ssems[b]
                ).wait()
            return carry

        return stage

    @pl.when(c == 0)
    def _():
        lax.fori_loop(0, _NSTG, make_stage(h2_t), 0)

    @pl.when(c == 1)
    def _():
        lax.fori_loop(0, _NSTG, make_stage(e2_t), 0)

    plsc.subcore_barrier()

    @pl.when(c == 0)
    def _():
        pltpu.sync_copy(acc_sh.at[pl.ds(s * _SLICE, _SLICE)],
                        s1_out.at[pl.ds(s * _SLICE, _SLICE)])

    @pl.when(c == 1)
    def _():
        pltpu.sync_copy(acc_sh.at[pl.ds(s * _SLICE, _SLICE)],
                        s2_out.at[pl.ds(s * _SLICE, _SLICE)])


_seg_call = functools.partial(
    pl.kernel,
    out_type=(
        jax.ShapeDtypeStruct((_NP, _D), jnp.float32),
        jax.ShapeDtypeStruct((_NP, _D), jnp.float32),
    ),
    mesh=_mesh,
    scratch_types=[
        pltpu.VMEM((_SROWS, _CH), jnp.int32),
        pltpu.VMEM((_SROWS, _CH), jnp.int32),
        pltpu.VMEM((_NB, _CH, _D), jnp.float32),
        pltpu.VMEM_SHARED((_NP, _D), jnp.float32),
        pltpu.SemaphoreType.DMA,
        pltpu.SemaphoreType.DMA,
        pltpu.SemaphoreType.DMA,
        pltpu.SemaphoreType.DMA,
        pltpu.SemaphoreType.DMA,
        pltpu.SemaphoreType.DMA,
        pltpu.SemaphoreType.DMA,
        pltpu.SemaphoreType.DMA,
    ],
)(_seg_body)


_BR = 1000


def _dense_body(feat, emb, wg, fm, d0, d1, h2, e2):
    di = lax.rsqrt(d0[...] + d1[...] + 1.0)
    h = jnp.dot(feat[...], wg[...], preferred_element_type=jnp.float32)
    h2[...] = h * di
    m = lax.dot_general(fm[...], fm[...], (((0,), (0,)), ((), ())),
                        preferred_element_type=jnp.float32)
    nrm = jnp.sqrt(jnp.sum(m * m)) + 1e-5
    e2[...] = jnp.dot(emb[...], m, preferred_element_type=jnp.float32) * (_KAPPA / nrm)


def _dense_call(features, embeddings, W_gcn, Fmat, d0, d1):
    return pl.pallas_call(
        _dense_body,
        grid=(_N // _BR,),
        in_specs=[
            pl.BlockSpec((_BR, _D), lambda i: (i, 0)),
            pl.BlockSpec((_BR, _D), lambda i: (i, 0)),
            pl.BlockSpec((_D, _D), lambda i: (0, 0)),
            pl.BlockSpec((_D, _D), lambda i: (0, 0)),
            pl.BlockSpec((_BR, 1), lambda i: (i, 0)),
            pl.BlockSpec((_BR, 1), lambda i: (i, 0)),
        ],
        out_specs=[
            pl.BlockSpec((_BR, _D), lambda i: (i, 0)),
            pl.BlockSpec((_BR, _D), lambda i: (i, 0)),
        ],
        out_shape=[
            jax.ShapeDtypeStruct((_N, _D), jnp.float32),
            jax.ShapeDtypeStruct((_N, _D), jnp.float32),
        ],
    )(features, embeddings, W_gcn, Fmat, d0, d1)


def _comb_body(feat, wg, d0, d1, s1, s2, b, out):
    di = lax.rsqrt(d0[...] + d1[...] + 1.0)
    h = jnp.dot(feat[...], wg[...], preferred_element_type=jnp.float32)
    out[...] = di * s1[...] + s2[...] + (di * di) * h + b[...]


def _comb_call(features, W_gcn, d0, d1, s1, s2, b2d):
    return pl.pallas_call(
        _comb_body,
        grid=(_N // _BR,),
        in_specs=[
            pl.BlockSpec((_BR, _D), lambda i: (i, 0)),
            pl.BlockSpec((_D, _D), lambda i: (0, 0)),
            pl.BlockSpec((_BR, 1), lambda i: (i, 0)),
            pl.BlockSpec((_BR, 1), lambda i: (i, 0)),
            pl.BlockSpec((_BR, _D), lambda i: (i, 0)),
            pl.BlockSpec((_BR, _D), lambda i: (i, 0)),
            pl.BlockSpec((1, _D), lambda i: (0, 0)),
        ],
        out_specs=pl.BlockSpec((_BR, _D), lambda i: (i, 0)),
        out_shape=jax.ShapeDtypeStruct((_N, _D), jnp.float32),
    )(features, W_gcn, d0, d1, s1, s2, b2d)


def kernel(features, sparse_adj, W_gcn, b_gcn, Fmat, embeddings):
    src4d = sparse_adj[0].astype(jnp.int32).reshape(_NSUB, _NSTG, _SROWS, _CH)
    dst4d = sparse_adj[1].astype(jnp.int32).reshape(_NSUB, _NSTG, _SROWS, _CH)
    zrow = jnp.zeros((_SLICE,), jnp.float32)

    degp = _deg_call(dst4d, zrow)
    d0 = degp[0].reshape(_NP, 1)
    d1 = degp[1].reshape(_NP, 1)
    h2, e2 = _dense_call(features, embeddings, W_gcn, Fmat, d0, d1)
    s1, s2 = _seg_call(h2, e2, src4d, dst4d)
    out = _comb_call(features, W_gcn, d0, d1, s1, s2, b_gcn.reshape(1, _D))
    return out


# speedup vs baseline: 27.8906x; 1.0131x over previous
import functools

import jax
import jax.numpy as jnp
from jax import lax
from jax.experimental import pallas as pl
from jax.experimental.pallas import tpu as pltpu
from jax.experimental.pallas import tpu_sc as plsc

_N = 10000
_D = 128
_E = 320000
_NP = 10240
_CH = 40
_ROWS = _E // _CH
_NSUB = 16
_NSTG = 5
_SROWS = _ROWS // _NSUB // _NSTG
_SLICE = _NP // _NSUB
_KAPPA = 0.95

_mesh = plsc.VectorSubcoreMesh(core_axis_name="c", subcore_axis_name="s")


def _deg_body(dst4d, zrow, deg_out, dstb, ones_v, deg_sh, sem):
    c = lax.axis_index("c")
    s = lax.axis_index("s")
    for k in range((_CH + 15) // 16):
        ones_v[pl.ds(min(16 * k, _CH - 16), 16)] = jnp.full((16,), 1.0,
                                                           jnp.float32)
    pltpu.sync_copy(zrow, deg_sh.at[pl.ds(s * _SLICE, _SLICE)])
    for st in range(_NSTG):
        pltpu.sync_copy(dst4d.at[s, st], dstb.at[st])
    plsc.subcore_barrier()
    half = _SROWS // 2

    def grp(g, carry):
        st = g // (half // 5)
        gg = g % (half // 5)
        descs = [
            pltpu.async_copy(
                ones_v, deg_sh.at[dstb.at[st, c * half + gg * 5 + b]], sem,
                add=True,
            )
            for b in range(5)
        ]
        for d in descs:
            d.wait()
        return carry

    lax.fori_loop(0, _NSTG * (half // 5), grp, 0)
    plsc.subcore_barrier()
    pltpu.sync_copy(deg_sh.at[pl.ds(s * _SLICE, _SLICE)],
                    deg_out.at[c, pl.ds(s * _SLICE, _SLICE)])


_deg_call = functools.partial(
    pl.kernel,
    out_type=jax.ShapeDtypeStruct((2, _NP), jnp.float32),
    mesh=_mesh,
    scratch_types=[
        pltpu.VMEM((_NSTG, _SROWS, _CH), jnp.int32),
        pltpu.VMEM((_CH,), jnp.float32),
        pltpu.VMEM_SHARED((_NP,), jnp.float32),
        pltpu.SemaphoreType.DMA,
    ],
)(_deg_body)


_NB = 4


def _seg_body(h2_t, e2_t, src4d, dst4d, s1_out, s2_out,
              srcb, dstb, rows, acc_sh, g0, g1, g2, g3, s0, s1sem, s2sem, s3sem):
    c = lax.axis_index("c")
    s = lax.axis_index("s")
    gsems = [g0, g1, g2, g3]
    ssems = [s0, s1sem, s2sem, s3sem]

    def zr(r, carry):
        for k in range(_D // 16):
            rows[0, r, pl.ds(16 * k, 16)] = jnp.zeros((16,), jnp.float32)
        return carry

    lax.fori_loop(0, _CH, zr, 0)
    for k in range(_SLICE // _CH):
        pltpu.sync_copy(rows.at[0],
                        acc_sh.at[pl.ds(s * _SLICE + k * _CH, _CH)])
    plsc.subcore_barrier()

    def make_stage(table):
        def stage(st, carry):
            pltpu.sync_copy(src4d.at[s, st], srcb)
            pltpu.sync_copy(dst4d.at[s, st], dstb)

            def grp(g, carry2):
                gd = []
                for b in range(_NB):
                    @pl.when(g > 0)
                    def _(b=b):
                        pltpu.make_async_copy(
                            table.at[pl.ds(0, _CH)], rows.at[b], ssems[b]
                        ).wait()
                    gd.append(
                        pltpu.async_copy(
                            table.at[srcb.at[g * _NB + b]], rows.at[b], gsems[b]
                        )
                    )
                for b in range(_NB):
                    gd[b].wait()
                    pltpu.async_copy(
                        rows.at[b], acc_sh.at[dstb.at[g * _NB + b]], ssems[b],
                        add=True,
                    )
                return carry2

            lax.fori_loop(0, _SROWS // _NB, grp, 0)
            for b in range(_NB):
                pltpu.make_async_copy(
                    table.at[pl.ds(0, _CH)], rows.at[b], ssems[b]
                ).wait()
            return carry

        return stage

    @pl.when(c == 0)
    def _():
        lax.fori_loop(0, _NSTG, make_stage(h2_t), 0)

    @pl.when(c == 1)
    def _():
        lax.fori_loop(0, _NSTG, make_stage(e2_t), 0)

    plsc.subcore_barrier()

    @pl.when(c == 0)
    def _():
        pltpu.sync_copy(acc_sh.at[pl.ds(s * _SLICE, _SLICE)],
                        s1_out.at[pl.ds(s * _SLICE, _SLICE)])

    @pl.when(c == 1)
    def _():
        pltpu.sync_copy(acc_sh.at[pl.ds(s * _SLICE, _SLICE)],
                        s2_out.at[pl.ds(s * _SLICE, _SLICE)])


_seg_call = functools.partial(
    pl.kernel,
    out_type=(
        jax.ShapeDtypeStruct((_NP, _D), jnp.float32),
        jax.ShapeDtypeStruct((_NP, _D), jnp.float32),
    ),
    mesh=_mesh,
    scratch_types=[
        pltpu.VMEM((_SROWS, _CH), jnp.int32),
        pltpu.VMEM((_SROWS, _CH), jnp.int32),
        pltpu.VMEM((_NB, _CH, _D), jnp.float32),
        pltpu.VMEM_SHARED((_NP, _D), jnp.float32),
        pltpu.SemaphoreType.DMA,
        pltpu.SemaphoreType.DMA,
        pltpu.SemaphoreType.DMA,
        pltpu.SemaphoreType.DMA,
        pltpu.SemaphoreType.DMA,
        pltpu.SemaphoreType.DMA,
        pltpu.SemaphoreType.DMA,
        pltpu.SemaphoreType.DMA,
    ],
)(_seg_body)


_BR = 1000


def _dense_body(feat, emb, wg, fm, d0, d1, h2, e2):
    di = lax.rsqrt(d0[...] + d1[...] + 1.0)
    h = jnp.dot(feat[...], wg[...], preferred_element_type=jnp.float32)
    h2[...] = h * di
    m = lax.dot_general(fm[...], fm[...], (((0,), (0,)), ((), ())),
                        preferred_element_type=jnp.float32)
    nrm = jnp.sqrt(jnp.sum(m * m)) + 1e-5
    e2[...] = jnp.dot(emb[...], m, preferred_element_type=jnp.float32) * (_KAPPA / nrm)


def _dense_call(features, embeddings, W_gcn, Fmat, d0, d1):
    return pl.pallas_call(
        _dense_body,
        grid=(_N // _BR,),
        in_specs=[
            pl.BlockSpec((_BR, _D), lambda i: (i, 0)),
            pl.BlockSpec((_BR, _D), lambda i: (i, 0)),
            pl.BlockSpec((_D, _D), lambda i: (0, 0)),
            pl.BlockSpec((_D, _D), lambda i: (0, 0)),
            pl.BlockSpec((_BR, 1), lambda i: (i, 0)),
            pl.BlockSpec((_BR, 1), lambda i: (i, 0)),
        ],
        out_specs=[
            pl.BlockSpec((_BR, _D), lambda i: (i, 0)),
            pl.BlockSpec((_BR, _D), lambda i: (i, 0)),
        ],
        out_shape=[
            jax.ShapeDtypeStruct((_N, _D), jnp.float32),
            jax.ShapeDtypeStruct((_N, _D), jnp.float32),
        ],
    )(features, embeddings, W_gcn, Fmat, d0, d1)


def _comb_body(feat, wg, d0, d1, s1, s2, b, out):
    di = lax.rsqrt(d0[...] + d1[...] + 1.0)
    h = jnp.dot(feat[...], wg[...], preferred_element_type=jnp.float32)
    out[...] = di * s1[...] + s2[...] + (di * di) * h + b[...]


def _comb_call(features, W_gcn, d0, d1, s1, s2, b2d):
    return pl.pallas_call(
        _comb_body,
        grid=(_N // _BR,),
        in_specs=[
            pl.BlockSpec((_BR, _D), lambda i: (i, 0)),
            pl.BlockSpec((_D, _D), lambda i: (0, 0)),
            pl.BlockSpec((_BR, 1), lambda i: (i, 0)),
            pl.BlockSpec((_BR, 1), lambda i: (i, 0)),
            pl.BlockSpec((_BR, _D), lambda i: (i, 0)),
            pl.BlockSpec((_BR, _D), lambda i: (i, 0)),
            pl.BlockSpec((1, _D), lambda i: (0, 0)),
        ],
        out_specs=pl.BlockSpec((_BR, _D), lambda i: (i, 0)),
        out_shape=jax.ShapeDtypeStruct((_N, _D), jnp.float32),
    )(features, W_gcn, d0, d1, s1, s2, b2d)


def kernel(features, sparse_adj, W_gcn, b_gcn, Fmat, embeddings):
    src4d = sparse_adj[0].astype(jnp.int32).reshape(_NSUB, _NSTG, _SROWS, _CH)
    dst4d = sparse_adj[1].astype(jnp.int32).reshape(_NSUB, _NSTG, _SROWS, _CH)
    zrow = jnp.zeros((_SLICE,), jnp.float32)

    degp = _deg_call(dst4d, zrow)
    d0 = degp[0].reshape(_NP, 1)
    d1 = degp[1].reshape(_NP, 1)
    h2, e2 = _dense_call(features, embeddings, W_gcn, Fmat, d0, d1)
    s1, s2 = _seg_call(h2, e2, src4d, dst4d)
    out = _comb_call(features, W_gcn, d0, d1, s1, s2, b_gcn.reshape(1, _D))
    return out


# speedup vs baseline: 28.1827x; 1.0105x over previous
import functools

import jax
import jax.numpy as jnp
from jax import lax
from jax.experimental import pallas as pl
from jax.experimental.pallas import tpu as pltpu
from jax.experimental.pallas import tpu_sc as plsc

_N = 10000
_D = 128
_E = 320000
_NP = 10240
_CH = 40
_ROWS = _E // _CH
_NSUB = 16
_NSTG = 5
_SROWS = _ROWS // _NSUB // _NSTG
_SLICE = _NP // _NSUB
_KAPPA = 0.95

_mesh = plsc.VectorSubcoreMesh(core_axis_name="c", subcore_axis_name="s")


_CHA = 40
_ASTG = 5
_ASROWS = _E // _CHA // _NSUB // _ASTG


def _deg_body(dst4d, zrow, deg_out, dstb, ones_v, deg_sh, sem):
    c = lax.axis_index("c")
    s = lax.axis_index("s")
    for k in range((_CHA + 15) // 16):
        ones_v[pl.ds(min(16 * k, _CHA - 16), 16)] = jnp.full((16,), 1.0,
                                                            jnp.float32)
    pltpu.sync_copy(zrow, deg_sh.at[pl.ds(s * _SLICE, _SLICE)])
    for st in range(_ASTG):
        pltpu.sync_copy(dst4d.at[s, st], dstb.at[st])
    plsc.subcore_barrier()
    half = _ASROWS // 2

    def grp(g, carry):
        st = g // (half // 10)
        gg = g % (half // 10)
        descs = [
            pltpu.async_copy(
                ones_v, deg_sh.at[dstb.at[st, c * half + gg * 10 + b]], sem,
                add=True,
            )
            for b in range(10)
        ]
        for d in descs:
            d.wait()
        return carry

    lax.fori_loop(0, _ASTG * (half // 10), grp, 0)
    plsc.subcore_barrier()
    pltpu.sync_copy(deg_sh.at[pl.ds(s * _SLICE, _SLICE)],
                    deg_out.at[c, pl.ds(s * _SLICE, _SLICE)])


_deg_call = functools.partial(
    pl.kernel,
    out_type=jax.ShapeDtypeStruct((2, _NP), jnp.float32),
    mesh=_mesh,
    scratch_types=[
        pltpu.VMEM((_ASTG, _ASROWS, _CHA), jnp.int32),
        pltpu.VMEM((_CHA,), jnp.float32),
        pltpu.VMEM_SHARED((_NP,), jnp.float32),
        pltpu.SemaphoreType.DMA,
    ],
)(_deg_body)


_NB = 4


def _seg_body(h2_t, e2_t, src4d, dst4d, s1_out, s2_out,
              srcb, dstb, rows, acc_sh, g0, g1, g2, g3,
              s0, s1sem, s2sem, s3sem):
    c = lax.axis_index("c")
    s = lax.axis_index("s")
    gsems = [g0, g1, g2, g3]
    ssems = [s0, s1sem, s2sem, s3sem]

    def zr(r, carry):
        for k in range(_D // 16):
            rows[0, r, pl.ds(16 * k, 16)] = jnp.zeros((16,), jnp.float32)
        return carry

    lax.fori_loop(0, _CH, zr, 0)
    zdescs = [
        pltpu.async_copy(rows.at[0],
                         acc_sh.at[pl.ds(s * _SLICE + k * _CH, _CH)], g0)
        for k in range(_SLICE // _CH)
    ]
    for d in zdescs:
        d.wait()
    plsc.subcore_barrier()

    def make_stage(table):
        def stage(st, carry):
            pltpu.sync_copy(src4d.at[s, st], srcb)
            pltpu.sync_copy(dst4d.at[s, st], dstb)

            def grp(g, carry2):
                gd = []
                for b in range(_NB):
                    @pl.when(g > 0)
                    def _(b=b):
                        pltpu.make_async_copy(
                            table.at[pl.ds(0, _CH)], rows.at[b], ssems[b]
                        ).wait()
                    gd.append(
                        pltpu.async_copy(
                            table.at[srcb.at[g * _NB + b]], rows.at[b], gsems[b]
                        )
                    )
                for b in range(_NB):
                    gd[b].wait()
                    pltpu.async_copy(
                        rows.at[b], acc_sh.at[dstb.at[g * _NB + b]], ssems[b],
                        add=True,
                    )
                return carry2

            lax.fori_loop(0, _SROWS // _NB, grp, 0)
            for b in range(_NB):
                pltpu.make_async_copy(
                    table.at[pl.ds(0, _CH)], rows.at[b], ssems[b]
                ).wait()
            return carry

        return stage

    @pl.when(c == 0)
    def _():
        lax.fori_loop(0, _NSTG, make_stage(h2_t), 0)

    @pl.when(c == 1)
    def _():
        lax.fori_loop(0, _NSTG, make_stage(e2_t), 0)

    plsc.subcore_barrier()

    @pl.when(c == 0)
    def _():
        pltpu.sync_copy(acc_sh.at[pl.ds(s * _SLICE, _SLICE)],
                        s1_out.at[pl.ds(s * _SLICE, _SLICE)])

    @pl.when(c == 1)
    def _():
        pltpu.sync_copy(acc_sh.at[pl.ds(s * _SLICE, _SLICE)],
                        s2_out.at[pl.ds(s * _SLICE, _SLICE)])


_seg_call = functools.partial(
    pl.kernel,
    out_type=(
        jax.ShapeDtypeStruct((_NP, _D), jnp.float32),
        jax.ShapeDtypeStruct((_NP, _D), jnp.float32),
    ),
    mesh=_mesh,
    scratch_types=[
        pltpu.VMEM((_SROWS, _CH), jnp.int32),
        pltpu.VMEM((_SROWS, _CH), jnp.int32),
        pltpu.VMEM((_NB, _CH, _D), jnp.float32),
        pltpu.VMEM_SHARED((_NP, _D), jnp.float32),
    ] + [pltpu.SemaphoreType.DMA] * 8,
)(_seg_body)


_BR = 1000


def _dense_body(feat, emb, wg, fm, d0, d1, h2, e2):
    di = lax.rsqrt(d0[...] + d1[...] + 1.0)
    h = jnp.dot(feat[...], wg[...], preferred_element_type=jnp.float32)
    h2[...] = h * di
    m = lax.dot_general(fm[...], fm[...], (((0,), (0,)), ((), ())),
                        preferred_element_type=jnp.float32)
    nrm = jnp.sqrt(jnp.sum(m * m)) + 1e-5
    e2[...] = jnp.dot(emb[...], m, preferred_element_type=jnp.float32) * (_KAPPA / nrm)


def _dense_call(features, embeddings, W_gcn, Fmat, d0, d1):
    return pl.pallas_call(
        _dense_body,
        grid=(_N // _BR,),
        in_specs=[
            pl.BlockSpec((_BR, _D), lambda i: (i, 0)),
            pl.BlockSpec((_BR, _D), lambda i: (i, 0)),
            pl.BlockSpec((_D, _D), lambda i: (0, 0)),
            pl.BlockSpec((_D, _D), lambda i: (0, 0)),
            pl.BlockSpec((_BR, 1), lambda i: (i, 0)),
            pl.BlockSpec((_BR, 1), lambda i: (i, 0)),
        ],
        out_specs=[
            pl.BlockSpec((_BR, _D), lambda i: (i, 0)),
            pl.BlockSpec((_BR, _D), lambda i: (i, 0)),
        ],
        out_shape=[
            jax.ShapeDtypeStruct((_N, _D), jnp.float32),
            jax.ShapeDtypeStruct((_N, _D), jnp.float32),
        ],
    )(features, embeddings, W_gcn, Fmat, d0, d1)


def _comb_body(feat, wg, d0, d1, s1, s2, b, out):
    di = lax.rsqrt(d0[...] + d1[...] + 1.0)
    h = jnp.dot(feat[...], wg[...], preferred_element_type=jnp.float32)
    out[...] = di * s1[...] + s2[...] + (di * di) * h + b[...]


def _comb_call(features, W_gcn, d0, d1, s1, s2, b2d):
    return pl.pallas_call(
        _comb_body,
        grid=(_N // _BR,),
        in_specs=[
            pl.BlockSpec((_BR, _D), lambda i: (i, 0)),
            pl.BlockSpec((_D, _D), lambda i: (0, 0)),
            pl.BlockSpec((_BR, 1), lambda i: (i, 0)),
            pl.BlockSpec((_BR, 1), lambda i: (i, 0)),
            pl.BlockSpec((_BR, _D), lambda i: (i, 0)),
            pl.BlockSpec((_BR, _D), lambda i: (i, 0)),
            pl.BlockSpec((1, _D), lambda i: (0, 0)),
        ],
        out_specs=pl.BlockSpec((_BR, _D), lambda i: (i, 0)),
        out_shape=jax.ShapeDtypeStruct((_N, _D), jnp.float32),
    )(features, W_gcn, d0, d1, s1, s2, b2d)


def kernel(features, sparse_adj, W_gcn, b_gcn, Fmat, embeddings):
    src4d = sparse_adj[0].astype(jnp.int32).reshape(_NSUB, _NSTG, _SROWS, _CH)
    dst4d = sparse_adj[1].astype(jnp.int32).reshape(_NSUB, _NSTG, _SROWS, _CH)
    dst4d_a = sparse_adj[1].astype(jnp.int32).reshape(_NSUB, _ASTG, _ASROWS,
                                                      _CHA)
    zrow = jnp.zeros((_SLICE,), jnp.float32)

    degp = _deg_call(dst4d_a, zrow)
    d0 = degp[0].reshape(_NP, 1)
    d1 = degp[1].reshape(_NP, 1)
    h2, e2 = _dense_call(features, embeddings, W_gcn, Fmat, d0, d1)
    s1, s2 = _seg_call(h2, e2, src4d, dst4d)
    out = _comb_call(features, W_gcn, d0, d1, s1, s2, b_gcn.reshape(1, _D))
    return out
